# baseline (speedup 1.0000x reference)
import functools

import jax
import jax.numpy as jnp
from jax import lax
from jax.experimental import pallas as pl
from jax.experimental.pallas import tpu as pltpu
from jax.experimental.pallas import tpu_sc as plsc

B = 16384
D = 32
L = 16
NC = 2
NS = 16
NW = NC * NS
BPW = B // NW
CHUNK = 128
NCHUNK = BPW // CHUNK
NBLK = BPW // L

_mesh = plsc.VectorSubcoreMesh(core_axis_name="c", subcore_axis_name="s")


@functools.partial(
    pl.kernel,
    out_type=jax.ShapeDtypeStruct((B,), jnp.float32),
    mesh=_mesh,
    compiler_params=pltpu.CompilerParams(needs_layout_passes=False,
                                         use_tc_tiling_on_sc=False),
    scratch_types=[
        pltpu.VMEM((BPW,), jnp.int32),
        pltpu.VMEM((BPW,), jnp.int32),
        pltpu.VMEM((BPW, D), jnp.float32),
        pltpu.VMEM((BPW, D), jnp.float32),
        pltpu.VMEM((BPW,), jnp.float32),
        pltpu.VMEM((BPW,), jnp.float32),
        pltpu.VMEM((BPW,), jnp.float32),
        pltpu.SemaphoreType.DMA,
    ],
)
def _mf_score(uid_hbm, iid_hbm, utab_hbm, itab_hbm, ubias_hbm, ibias_hbm,
              out_hbm, idx_u, idx_i, rows_u, rows_i, bias_u, bias_i,
              out_v, sem):
    wid = lax.axis_index("s") * NC + lax.axis_index("c")
    base = wid * BPW

    pltpu.sync_copy(uid_hbm.at[pl.ds(base, BPW)], idx_u)
    pltpu.sync_copy(iid_hbm.at[pl.ds(base, BPW)], idx_i)

    copies = []
    for c in range(NCHUNK):
        s = pl.ds(c * CHUNK, CHUNK)
        copies.append(pltpu.async_copy(utab_hbm.at[idx_u.at[s]], rows_u.at[s], sem))
        copies.append(pltpu.async_copy(itab_hbm.at[idx_i.at[s]], rows_i.at[s], sem))
        copies.append(pltpu.async_copy(ubias_hbm.at[idx_u.at[s]], bias_u.at[s], sem))
        copies.append(pltpu.async_copy(ibias_hbm.at[idx_i.at[s]], bias_i.at[s], sem))
    for cp in copies:
        cp.wait()

    def blk_fn(j, _):
        row = lax.iota(jnp.int32, L) + j * L
        acc = bias_u[pl.ds(j * L, L)] + bias_i[pl.ds(j * L, L)]
        for d in range(D):
            col = jnp.full((L,), d, jnp.int32)
            acc = acc + (plsc.load_gather(rows_u, [row, col])
                         * plsc.load_gather(rows_i, [row, col]))
        out_v[pl.ds(j * L, L)] = acc
        return 0

    lax.fori_loop(0, NBLK, blk_fn, 0)
    pltpu.sync_copy(out_v, out_hbm.at[pl.ds(base, BPW)])


def kernel(user_ids, item_ids, user_table, item_table, user_bias, item_bias):
    return _mf_score(user_ids.astype(jnp.int32), item_ids.astype(jnp.int32),
                     user_table, item_table,
                     user_bias.reshape(-1), item_bias.reshape(-1))
